# baseline (speedup 1.0000x reference)
import functools

import jax
import jax.numpy as jnp
from jax.experimental import pallas as pl
from jax.experimental.pallas import tpu as pltpu

_NBLK = 1024
_NCHUNK = 4


def _merge3(x, y):
    x0, x1, x2 = x
    y0, y1, y2 = y
    z0 = jnp.minimum(x0, y0)
    z1 = jnp.minimum(jnp.maximum(x0, y0), jnp.minimum(x1, y1))
    z2 = jnp.minimum(jnp.minimum(x2, y2),
                     jnp.minimum(jnp.maximum(x1, y0), jnp.maximum(x0, y1)))
    return z0, z1, z2


def _chunk_top3(d2):
    n, c = d2.shape
    h = c // 2
    x, y = d2[:, :h], d2[:, h:]
    t0 = jnp.minimum(x, y)
    t1 = jnp.maximum(x, y)
    q = h // 2
    x0, x1 = t0[:, :q], t1[:, :q]
    y0, y1 = t0[:, q:], t1[:, q:]
    z0 = jnp.minimum(x0, y0)
    z1 = jnp.minimum(jnp.maximum(x0, y0), jnp.minimum(x1, y1))
    z2 = jnp.minimum(jnp.maximum(x1, y0), jnp.maximum(x0, y1))
    return z0, z1, z2


def _knn_softmin_kernel(nsteps, e_ref, c_ref, out_ref, cent_ref, acc_ref,
                        feat_ref):
    p, d = c_ref.shape
    ch = p // _NCHUNK
    i = pl.program_id(0)

    @pl.when(i == 0)
    def _init_cent():
        ones_half = jnp.full((1, d), 0.5, dtype=jnp.float32)
        for k in range(_NCHUNK):
            sl = pl.ds(k * ch, ch)
            c = c_ref[sl, :]
            cent_ref[:, sl] = jax.lax.dot_general(
                ones_half, c * c,
                dimension_numbers=(((1,), (1,)), ((), ())),
                preferred_element_type=jnp.float32,
            )

    def _tail():
        a0t = acc_ref[0].T
        a1t = acc_ref[1].T
        a2t = acc_ref[2].T
        w = a0t.shape[0]
        sub = jax.lax.broadcasted_iota(jnp.int32, a0t.shape, 0)

        m0 = jnp.min(a0t, axis=0, keepdims=True)
        i0 = jnp.min(jnp.where(a0t == m0, sub, w), axis=0, keepdims=True)
        b0 = jnp.where(sub == i0, a1t, a0t)

        m1 = jnp.min(b0, axis=0, keepdims=True)
        i1 = jnp.min(jnp.where(b0 == m1, sub, w), axis=0, keepdims=True)
        rep = jnp.where(i1 == i0, a2t, a1t)
        c0 = jnp.where(sub == i1, rep, b0)

        m2 = jnp.min(c0, axis=0, keepdims=True)

        fc = feat_ref[...]
        d0 = jnp.sqrt(2.0 * m0 + fc)
        d1 = jnp.sqrt(2.0 * m1 + fc)
        d2v = jnp.sqrt(2.0 * m2 + fc)

        s0 = 1.0 / (1.0 + jnp.exp(d0 - d1) + jnp.exp(d0 - d2v))
        out_ref[...] = (s0 * d0).reshape(1, 1, d0.shape[1])

    _tail()

    @pl.when(i < nsteps)
    def _compute():
        e = e_ref[...]
        ones_row = jnp.ones((1, d), dtype=jnp.float32)
        feat_ref[...] = jax.lax.dot_general(
            e * e, ones_row,
            dimension_numbers=(((1,), (1,)), ((), ())),
            preferred_element_type=jnp.float32,
        ).reshape(1, e.shape[0])

        acc = None
        for k in range(_NCHUNK):
            sl = pl.ds(k * ch, ch)
            f = jax.lax.dot_general(
                e, c_ref[sl, :],
                dimension_numbers=(((1,), (1,)), ((), ())),
                preferred_element_type=jnp.float32,
            )
            s = cent_ref[:, sl] - f
            tri = _chunk_top3(s)
            acc = tri if acc is None else _merge3(acc, tri)

        acc_ref[0] = acc[0]
        acc_ref[1] = acc[1]
        acc_ref[2] = acc[2]


@jax.jit
def kernel(embeds, centroids):
    b, n, d = embeds.shape
    p = centroids.shape[0]
    bn = b * n
    e2 = embeds.reshape(bn, d)
    nsteps = bn // _NBLK
    w = (p // _NCHUNK) // 4

    out = pl.pallas_call(
        functools.partial(_knn_softmin_kernel, nsteps),
        grid=(nsteps + 1,),
        in_specs=[
            pl.BlockSpec((_NBLK, d), lambda i: (jnp.minimum(i, nsteps - 1), 0)),
            pl.BlockSpec((p, d), lambda i: (0, 0)),
        ],
        out_specs=pl.BlockSpec((1, 1, _NBLK),
                               lambda i: (jnp.maximum(i - 1, 0), 0, 0)),
        out_shape=jax.ShapeDtypeStruct((nsteps, 1, _NBLK), jnp.float32),
        scratch_shapes=[
            pltpu.VMEM((1, p), jnp.float32),
            pltpu.VMEM((3, _NBLK, w), jnp.float32),
            pltpu.VMEM((1, _NBLK), jnp.float32),
        ],
        compiler_params=pltpu.CompilerParams(
            dimension_semantics=("arbitrary",),
        ),
    )(e2, centroids)

    h = 32
    return out.reshape(b, 1, h, n // h)


# speedup vs baseline: 1.0026x; 1.0026x over previous
import functools

import jax
import jax.numpy as jnp
from jax.experimental import pallas as pl
from jax.experimental.pallas import tpu as pltpu

_NBLK = 2048
_NCHUNK = 4


def _merge3(x, y):
    x0, x1, x2 = x
    y0, y1, y2 = y
    z0 = jnp.minimum(x0, y0)
    z1 = jnp.minimum(jnp.maximum(x0, y0), jnp.minimum(x1, y1))
    z2 = jnp.minimum(jnp.minimum(x2, y2),
                     jnp.minimum(jnp.maximum(x1, y0), jnp.maximum(x0, y1)))
    return z0, z1, z2


def _chunk_top3(d2):
    n, c = d2.shape
    h = c // 2
    x, y = d2[:, :h], d2[:, h:]
    t0 = jnp.minimum(x, y)
    t1 = jnp.maximum(x, y)
    q = h // 2
    x0, x1 = t0[:, :q], t1[:, :q]
    y0, y1 = t0[:, q:], t1[:, q:]
    z0 = jnp.minimum(x0, y0)
    z1 = jnp.minimum(jnp.maximum(x0, y0), jnp.minimum(x1, y1))
    z2 = jnp.minimum(jnp.maximum(x1, y0), jnp.maximum(x0, y1))
    return z0, z1, z2


def _knn_softmin_kernel(e_ref, c_ref, out_ref, cent_ref):
    p, d = c_ref.shape
    ch = p // _NCHUNK

    @pl.when(pl.program_id(0) == 0)
    def _init_cent():
        ones_half = jnp.full((1, d), 0.5, dtype=jnp.float32)
        for k in range(_NCHUNK):
            sl = pl.ds(k * ch, ch)
            c = c_ref[sl, :]
            cent_ref[:, sl] = jax.lax.dot_general(
                ones_half, c * c,
                dimension_numbers=(((1,), (1,)), ((), ())),
                preferred_element_type=jnp.float32,
            )

    e = e_ref[...]
    ones_col = jnp.ones((1, d), dtype=jnp.float32)
    feat = jax.lax.dot_general(
        e * e, ones_col,
        dimension_numbers=(((1,), (1,)), ((), ())),
        preferred_element_type=jnp.float32,
    )

    acc = None
    for k in range(_NCHUNK):
        sl = pl.ds(k * ch, ch)
        f = jax.lax.dot_general(
            e, c_ref[sl, :],
            dimension_numbers=(((1,), (1,)), ((), ())),
            preferred_element_type=jnp.float32,
        )
        s = cent_ref[:, sl] - f
        tri = _chunk_top3(s)
        acc = tri if acc is None else _merge3(acc, tri)

    a0, a1, a2 = acc
    nrow = a0.shape[0]
    w = a0.shape[1]
    a0t = a0.T
    a1t = a1.T
    a2t = a2.T
    sub = jax.lax.broadcasted_iota(jnp.int32, a0t.shape, 0)

    m0 = jnp.min(a0t, axis=0, keepdims=True)
    i0 = jnp.min(jnp.where(a0t == m0, sub, w), axis=0, keepdims=True)
    b0 = jnp.where(sub == i0, a1t, a0t)

    m1 = jnp.min(b0, axis=0, keepdims=True)
    i1 = jnp.min(jnp.where(b0 == m1, sub, w), axis=0, keepdims=True)
    rep = jnp.where(i1 == i0, a2t, a1t)
    c0 = jnp.where(sub == i1, rep, b0)

    m2 = jnp.min(c0, axis=0, keepdims=True)

    fc = feat.reshape(1, nrow)
    d0 = jnp.sqrt(2.0 * m0 + fc)
    d1 = jnp.sqrt(2.0 * m1 + fc)
    d2v = jnp.sqrt(2.0 * m2 + fc)

    s0 = 1.0 / (1.0 + jnp.exp(d0 - d1) + jnp.exp(d0 - d2v))
    out_ref[...] = (s0 * d0).reshape(1, 1, nrow)


@jax.jit
def kernel(embeds, centroids):
    b, n, d = embeds.shape
    p = centroids.shape[0]
    bn = b * n
    e2 = embeds.reshape(bn, d)

    grid = (bn // _NBLK,)
    out = pl.pallas_call(
        _knn_softmin_kernel,
        grid=grid,
        in_specs=[
            pl.BlockSpec((_NBLK, d), lambda i: (i, 0)),
            pl.BlockSpec((p, d), lambda i: (0, 0)),
        ],
        out_specs=pl.BlockSpec((1, 1, _NBLK), lambda i: (i, 0, 0)),
        out_shape=jax.ShapeDtypeStruct((bn // _NBLK, 1, _NBLK), jnp.float32),
        scratch_shapes=[
            pltpu.VMEM((1, p), jnp.float32),
        ],
        compiler_params=pltpu.CompilerParams(
            dimension_semantics=("arbitrary",),
        ),
    )(e2, centroids)

    h = 32
    w = n // h
    return out.reshape(b, 1, h, w)


# speedup vs baseline: 1.0694x; 1.0667x over previous
import functools

import jax
import jax.numpy as jnp
from jax.experimental import pallas as pl
from jax.experimental.pallas import tpu as pltpu

_NBLK = 1024
_NCHUNK = 4


def _merge3(x, y):
    x0, x1, x2 = x
    y0, y1, y2 = y
    z0 = jnp.minimum(x0, y0)
    z1 = jnp.minimum(jnp.maximum(x0, y0), jnp.minimum(x1, y1))
    z2 = jnp.minimum(jnp.minimum(x2, y2),
                     jnp.minimum(jnp.maximum(x1, y0), jnp.maximum(x0, y1)))
    return z0, z1, z2


def _chunk_top3(d2):
    n, c = d2.shape
    h = c // 2
    x, y = d2[:, :h], d2[:, h:]
    t0 = jnp.minimum(x, y)
    t1 = jnp.maximum(x, y)
    q = h // 2
    x0, x1 = t0[:, :q], t1[:, :q]
    y0, y1 = t0[:, q:], t1[:, q:]
    z0 = jnp.minimum(x0, y0)
    z1 = jnp.minimum(jnp.maximum(x0, y0), jnp.minimum(x1, y1))
    z2 = jnp.minimum(jnp.maximum(x1, y0), jnp.maximum(x0, y1))
    return z0, z1, z2


def _knn_softmin_kernel(e_ref, c_ref, out_ref, cent_ref):
    p, d = c_ref.shape
    ch = p // _NCHUNK

    @pl.when(pl.program_id(0) == 0)
    def _init_cent():
        ones_half = jnp.full((1, d), 0.5, dtype=jnp.float32)
        for k in range(_NCHUNK):
            sl = pl.ds(k * ch, ch)
            c = c_ref[sl, :]
            cent_ref[:, sl] = jax.lax.dot_general(
                ones_half, c * c,
                dimension_numbers=(((1,), (1,)), ((), ())),
                preferred_element_type=jnp.float32,
            )

    e = e_ref[...]
    ones_col = jnp.ones((1, d), dtype=jnp.float32)
    feat = jax.lax.dot_general(
        e * e, ones_col,
        dimension_numbers=(((1,), (1,)), ((), ())),
        preferred_element_type=jnp.float32,
    )

    acc = None
    for k in range(_NCHUNK):
        sl = pl.ds(k * ch, ch)
        f = jax.lax.dot_general(
            e, c_ref[sl, :],
            dimension_numbers=(((1,), (1,)), ((), ())),
            preferred_element_type=jnp.float32,
        )
        s = cent_ref[:, sl] - f
        tri = _chunk_top3(s)
        acc = tri if acc is None else _merge3(acc, tri)

    a0, a1, a2 = acc
    nrow = a0.shape[0]
    w = a0.shape[1]
    a0t = a0.T
    a1t = a1.T
    a2t = a2.T
    sub = jax.lax.broadcasted_iota(jnp.int32, a0t.shape, 0)

    m0 = jnp.min(a0t, axis=0, keepdims=True)
    i0 = jnp.min(jnp.where(a0t == m0, sub, w), axis=0, keepdims=True)
    b0 = jnp.where(sub == i0, a1t, a0t)

    m1 = jnp.min(b0, axis=0, keepdims=True)
    i1 = jnp.min(jnp.where(b0 == m1, sub, w), axis=0, keepdims=True)
    rep = jnp.where(i1 == i0, a2t, a1t)
    c0 = jnp.where(sub == i1, rep, b0)

    m2 = jnp.min(c0, axis=0, keepdims=True)

    fc = feat.reshape(1, nrow)
    d0 = jnp.sqrt(2.0 * m0 + fc)
    d1 = jnp.sqrt(2.0 * m1 + fc)
    d2v = jnp.sqrt(2.0 * m2 + fc)

    s0 = 1.0 / (1.0 + jnp.exp(d0 - d1) + jnp.exp(d0 - d2v))
    out_ref[...] = (s0 * d0).reshape(1, 1, nrow)


@jax.jit
def kernel(embeds, centroids):
    b, n, d = embeds.shape
    p = centroids.shape[0]
    bn = b * n
    e2 = embeds.reshape(bn, d)

    grid = (bn // _NBLK,)
    out = pl.pallas_call(
        _knn_softmin_kernel,
        grid=grid,
        in_specs=[
            pl.BlockSpec((_NBLK, d), lambda i: (i, 0)),
            pl.BlockSpec((p, d), lambda i: (0, 0)),
        ],
        out_specs=pl.BlockSpec((1, 1, _NBLK), lambda i: (i, 0, 0)),
        out_shape=jax.ShapeDtypeStruct((bn // _NBLK, 1, _NBLK), jnp.float32),
        scratch_shapes=[
            pltpu.VMEM((1, p), jnp.float32),
        ],
        compiler_params=pltpu.CompilerParams(
            dimension_semantics=("arbitrary",),
        ),
    )(e2, centroids)

    h = 32
    w = n // h
    return out.reshape(b, 1, h, w)
